# baseline (speedup 1.0000x reference)
import functools

import jax
import jax.numpy as jnp
from jax import lax
from jax.experimental import pallas as pl
from jax.experimental.pallas import tpu as pltpu
from jax.experimental.pallas import tpu_sc as plsc

N = 10000
D = 128
H = 32
T = 10
E = 320000

NC = 2
NS = 16
NW = NC * NS
EPW = E // NW
CH = 80
NCHUNK = EPW // CH
ROWS_PT = 640
NPAD = NS * ROWS_PT


def _mesh():
    return plsc.VectorSubcoreMesh(core_axis_name="c", subcore_axis_name="s")


@functools.partial(
    pl.kernel,
    out_type=jax.ShapeDtypeStruct((NC, NPAD), jnp.float32),
    mesh=_mesh(),
    compiler_params=pltpu.CompilerParams(use_tc_tiling_on_sc=False),
    scratch_types=[
        pltpu.VMEM((NCHUNK, CH), jnp.int32),
        pltpu.VMEM((CH,), jnp.float32),
        pltpu.VMEM((ROWS_PT,), jnp.float32),
        pltpu.VMEM_SHARED((NPAD,), jnp.float32),
        pltpu.SemaphoreType.DMA,
    ],
)
def _sc_degree(dst_hbm, out_hbm, idx_v, ones_v, buf_v, acc_sh, sem):
    cid = lax.axis_index("c")
    sid = lax.axis_index("s")
    wid = sid * NC + cid

    def fill(i, _):
        buf_v[pl.ds(i * 16, 16)] = jnp.zeros((16,), jnp.float32)
        return 0

    lax.fori_loop(0, ROWS_PT // 16, fill, 0)

    def fill1(i, _):
        ones_v[pl.ds(i * 16, 16)] = jnp.ones((16,), jnp.float32)
        return 0

    lax.fori_loop(0, CH // 16, fill1, 0)

    pltpu.sync_copy(buf_v, acc_sh.at[pl.ds(sid * ROWS_PT, ROWS_PT)])
    pltpu.sync_copy(dst_hbm.at[wid], idx_v)
    plsc.subcore_barrier()

    def body(j, _):
        pltpu.sync_copy(ones_v, acc_sh.at[idx_v.at[j]], add=True)
        return 0

    lax.fori_loop(0, NCHUNK, body, 0)
    plsc.subcore_barrier()

    pltpu.sync_copy(acc_sh.at[pl.ds(sid * ROWS_PT, ROWS_PT)], buf_v)
    pltpu.sync_copy(buf_v, out_hbm.at[cid, pl.ds(sid * ROWS_PT, ROWS_PT)])


@functools.partial(
    pl.kernel,
    out_type=jax.ShapeDtypeStruct((NC, NPAD, H), jnp.float32),
    mesh=_mesh(),
    compiler_params=pltpu.CompilerParams(use_tc_tiling_on_sc=False),
    scratch_types=[
        pltpu.VMEM((NCHUNK, CH), jnp.int32),
        pltpu.VMEM((NCHUNK, CH), jnp.int32),
        pltpu.VMEM((CH, H), jnp.float32),
        pltpu.VMEM((ROWS_PT, H), jnp.float32),
        pltpu.VMEM_SHARED((NPAD, H), jnp.float32),
        pltpu.SemaphoreType.DMA,
    ],
)
def _sc_message(src_hbm, dst_hbm, y_hbm, out_hbm, srcv, dstv, rows, buf, acc_sh, sem):
    cid = lax.axis_index("c")
    sid = lax.axis_index("s")
    wid = sid * NC + cid

    def fill(i, _):
        buf[i, pl.ds(0, 16)] = jnp.zeros((16,), jnp.float32)
        buf[i, pl.ds(16, 16)] = jnp.zeros((16,), jnp.float32)
        return 0

    lax.fori_loop(0, ROWS_PT, fill, 0)

    pltpu.sync_copy(buf, acc_sh.at[pl.ds(sid * ROWS_PT, ROWS_PT)])
    pltpu.sync_copy(src_hbm.at[wid], srcv)
    pltpu.sync_copy(dst_hbm.at[wid], dstv)
    plsc.subcore_barrier()

    def body(j, _):
        pltpu.async_copy(y_hbm.at[srcv.at[j]], rows, sem).wait()
        pltpu.sync_copy(rows, acc_sh.at[dstv.at[j]], add=True)
        return 0

    lax.fori_loop(0, NCHUNK, body, 0)
    plsc.subcore_barrier()

    pltpu.sync_copy(acc_sh.at[pl.ds(sid * ROWS_PT, ROWS_PT)], buf)
    pltpu.sync_copy(buf, out_hbm.at[cid, pl.ds(sid * ROWS_PT, ROWS_PT)])


def _tc_first(x, W1, deg0, deg1):
    def body(x_ref, w_ref, d0_ref, d1_ref, y_ref, dinv_ref):
        deg = d0_ref[...] + d1_ref[...] + 1.0
        dinv = lax.rsqrt(deg)
        xw = jnp.dot(x_ref[...], w_ref[...], preferred_element_type=jnp.float32)
        y_ref[...] = xw * dinv
        dinv_ref[...] = dinv

    return pl.pallas_call(
        body,
        out_shape=[
            jax.ShapeDtypeStruct((N, H), jnp.float32),
            jax.ShapeDtypeStruct((N, 1), jnp.float32),
        ],
    )(x, W1, deg0, deg1)


def _tc_mid(a0, a1, y, dinv, b, W2):
    def body(a0_ref, a1_ref, y_ref, dinv_ref, b_ref, w_ref, y2_ref):
        s = a0_ref[...] + a1_ref[...] + y_ref[...]
        h = jnp.maximum(s * dinv_ref[...] + b_ref[...], 0.0)
        hw = jnp.dot(h, w_ref[...], preferred_element_type=jnp.float32)
        y2_ref[...] = hw * dinv_ref[...]

    return pl.pallas_call(
        body,
        out_shape=jax.ShapeDtypeStruct((N, H), jnp.float32),
    )(a0, a1, y, dinv, b, W2)


def _tc_gates(a0, a1, y, dinv, b, W_ih4, bg4):
    def body(a0_ref, a1_ref, y_ref, dinv_ref, b_ref, w_ref, bg_ref,
             gi_ref, gf_ref, gg_ref, go_ref):
        s = a0_ref[...] + a1_ref[...] + y_ref[...]
        h = jnp.maximum(s * dinv_ref[...] + b_ref[...], 0.0)
        dn = (((1,), (1,)), ((), ()))
        for k, out in enumerate((gi_ref, gf_ref, gg_ref, go_ref)):
            out[...] = (
                lax.dot_general(h, w_ref[k], dn,
                                preferred_element_type=jnp.float32)
                + bg_ref[k]
            )

    sd = jax.ShapeDtypeStruct((N, H), jnp.float32)
    return pl.pallas_call(
        body,
        out_shape=[sd, sd, sd, sd],
    )(a0, a1, y, dinv, b, W_ih4, bg4)


def _tc_lstm(Gi, Gf, Gg, Go, W_hh4, WfcT, bfc):
    def body(gi_ref, gf_ref, gg_ref, go_ref, whh_ref, wfc_ref, bfc_ref,
             out_ref, hs_ref):
        wi = whh_ref[0]
        wf = whh_ref[1]
        wg = whh_ref[2]
        wo = whh_ref[3]
        dn = (((1,), (1,)), ((), ()))

        def step(t, carry):
            h, c = carry
            ui = lax.dot_general(h, wi, dn, precision=lax.Precision.DEFAULT,
                                 preferred_element_type=jnp.float32)
            uf = lax.dot_general(h, wf, dn, precision=lax.Precision.DEFAULT,
                                 preferred_element_type=jnp.float32)
            ug = lax.dot_general(h, wg, dn, precision=lax.Precision.DEFAULT,
                                 preferred_element_type=jnp.float32)
            uo = lax.dot_general(h, wo, dn, precision=lax.Precision.DEFAULT,
                                 preferred_element_type=jnp.float32)
            row = pl.ds(t, 1)
            i = 0.5 * jnp.tanh(0.5 * (gi_ref[row, :] + ui)) + 0.5
            f = 0.5 * jnp.tanh(0.5 * (gf_ref[row, :] + uf)) + 0.5
            g = jnp.tanh(gg_ref[row, :] + ug)
            o = 0.5 * jnp.tanh(0.5 * (go_ref[row, :] + uo)) + 0.5
            c = f * c + i * g
            h = o * jnp.tanh(c)
            hs_ref[row, :] = h
            return (h, c)

        h0 = jnp.zeros((1, H), jnp.float32)
        lax.fori_loop(0, N, step, (h0, h0))
        out_ref[...] = (
            jnp.dot(hs_ref[...], wfc_ref[...], preferred_element_type=jnp.float32)
            + bfc_ref[...]
        )

    return pl.pallas_call(
        body,
        out_shape=jax.ShapeDtypeStruct((N, T), jnp.float32),
        scratch_shapes=[pltpu.VMEM((N, H), jnp.float32)],
    )(Gi, Gf, Gg, Go, W_hh4, WfcT, bfc)


def kernel(x, edge_index, W1, b1, W2, b2, W_ih, W_hh, b_ih, b_hh, Wfc, bfc):
    src = edge_index[0].reshape(NW, NCHUNK, CH)
    dst = edge_index[1].reshape(NW, NCHUNK, CH)

    deg_parts = _sc_degree(dst)
    deg0 = deg_parts[0, :N].reshape(N, 1)
    deg1 = deg_parts[1, :N].reshape(N, 1)

    y1, dinv = _tc_first(x, W1, deg0, deg1)

    acc1 = _sc_message(src, dst, y1)
    y2 = _tc_mid(acc1[0, :N], acc1[1, :N], y1, dinv, b1.reshape(1, H), W2)

    acc2 = _sc_message(src, dst, y2)
    Gi, Gf, Gg, Go = _tc_gates(
        acc2[0, :N], acc2[1, :N], y2, dinv, b2.reshape(1, H),
        W_ih.reshape(4, H, H), (b_ih + b_hh).reshape(4, 1, H),
    )

    return _tc_lstm(Gi, Gf, Gg, Go, W_hh.reshape(4, H, H), Wfc.T,
                    bfc.reshape(1, T))


# speedup vs baseline: 1.0337x; 1.0337x over previous
import functools

import jax
import jax.numpy as jnp
from jax import lax
from jax.experimental import pallas as pl
from jax.experimental.pallas import tpu as pltpu
from jax.experimental.pallas import tpu_sc as plsc

N = 10000
D = 128
H = 32
T = 10
E = 320000

NC = 2
NS = 16
NW = NC * NS
EPW = E // NW
CH = 80
NCHUNK = EPW // CH
ROWS_PT = 640
NPAD = NS * ROWS_PT


def _mesh():
    return plsc.VectorSubcoreMesh(core_axis_name="c", subcore_axis_name="s")


@functools.partial(
    pl.kernel,
    out_type=jax.ShapeDtypeStruct((NC, NPAD), jnp.float32),
    mesh=_mesh(),
    compiler_params=pltpu.CompilerParams(use_tc_tiling_on_sc=False),
    scratch_types=[
        pltpu.VMEM((NCHUNK, CH), jnp.int32),
        pltpu.VMEM((CH,), jnp.float32),
        pltpu.VMEM((ROWS_PT,), jnp.float32),
        pltpu.VMEM_SHARED((NPAD,), jnp.float32),
        pltpu.SemaphoreType.DMA,
    ],
)
def _sc_degree(edges_hbm, out_hbm, idx_v, ones_v, buf_v, acc_sh, sem):
    cid = lax.axis_index("c")
    sid = lax.axis_index("s")
    wid = sid * NC + cid

    def fill(i, _):
        buf_v[pl.ds(i * 16, 16)] = jnp.zeros((16,), jnp.float32)
        return 0

    lax.fori_loop(0, ROWS_PT // 16, fill, 0)

    def fill1(i, _):
        ones_v[pl.ds(i * 16, 16)] = jnp.ones((16,), jnp.float32)
        return 0

    lax.fori_loop(0, CH // 16, fill1, 0)

    pltpu.sync_copy(buf_v, acc_sh.at[pl.ds(sid * ROWS_PT, ROWS_PT)])
    pltpu.sync_copy(edges_hbm.at[1, wid], idx_v)
    plsc.subcore_barrier()

    def body(j, _):
        pltpu.sync_copy(ones_v, acc_sh.at[idx_v.at[j]], add=True)
        return 0

    lax.fori_loop(0, NCHUNK, body, 0)
    plsc.subcore_barrier()

    pltpu.sync_copy(acc_sh.at[pl.ds(sid * ROWS_PT, ROWS_PT)], buf_v)
    pltpu.sync_copy(buf_v, out_hbm.at[cid, pl.ds(sid * ROWS_PT, ROWS_PT)])


@functools.partial(
    pl.kernel,
    out_type=jax.ShapeDtypeStruct((NC, NPAD, H), jnp.float32),
    mesh=_mesh(),
    compiler_params=pltpu.CompilerParams(use_tc_tiling_on_sc=False),
    scratch_types=[
        pltpu.VMEM((NCHUNK, CH), jnp.int32),
        pltpu.VMEM((NCHUNK, CH), jnp.int32),
        pltpu.VMEM((2, CH, H), jnp.float32),
        pltpu.VMEM((ROWS_PT, H), jnp.float32),
        pltpu.VMEM_SHARED((NPAD, H), jnp.float32),
        pltpu.SemaphoreType.DMA,
    ],
)
def _sc_message(edges_hbm, y_hbm, out_hbm, srcv, dstv, rows, buf, acc_sh, sem):
    cid = lax.axis_index("c")
    sid = lax.axis_index("s")
    wid = sid * NC + cid

    def fill(i, _):
        buf[i, pl.ds(0, 16)] = jnp.zeros((16,), jnp.float32)
        buf[i, pl.ds(16, 16)] = jnp.zeros((16,), jnp.float32)
        return 0

    lax.fori_loop(0, ROWS_PT, fill, 0)

    pltpu.sync_copy(buf, acc_sh.at[pl.ds(sid * ROWS_PT, ROWS_PT)])
    pltpu.sync_copy(edges_hbm.at[0, wid], srcv)
    pltpu.sync_copy(edges_hbm.at[1, wid], dstv)
    plsc.subcore_barrier()

    pltpu.async_copy(y_hbm.at[srcv.at[0]], rows.at[0], sem)

    def body(j, _):
        jm = j % 2
        pltpu.make_async_copy(y_hbm.at[srcv.at[0]], rows.at[jm], sem).wait()

        @pl.when(j + 1 < NCHUNK)
        def _():
            pltpu.async_copy(y_hbm.at[srcv.at[j + 1]], rows.at[1 - jm], sem)

        pltpu.sync_copy(rows.at[jm], acc_sh.at[dstv.at[j]], add=True)
        return 0

    lax.fori_loop(0, NCHUNK, body, 0)
    plsc.subcore_barrier()

    pltpu.sync_copy(acc_sh.at[pl.ds(sid * ROWS_PT, ROWS_PT)], buf)
    pltpu.sync_copy(buf, out_hbm.at[cid, pl.ds(sid * ROWS_PT, ROWS_PT)])


def _tc_first(x, W1, deg_parts):
    def body(x_ref, w_ref, d_ref, y_ref, dinv_ref):
        nsl = pl.ds(0, N)
        deg = d_ref[0, nsl, :] + d_ref[1, nsl, :] + 1.0
        dinv = lax.rsqrt(deg)
        xw = jnp.dot(x_ref[...], w_ref[...], preferred_element_type=jnp.float32)
        y_ref[...] = xw * dinv
        dinv_ref[...] = dinv

    return pl.pallas_call(
        body,
        out_shape=[
            jax.ShapeDtypeStruct((N, H), jnp.float32),
            jax.ShapeDtypeStruct((N, 1), jnp.float32),
        ],
    )(x, W1, deg_parts)


def _tc_mid(acc, y, dinv, b, W2):
    def body(a_ref, y_ref, dinv_ref, b_ref, w_ref, y2_ref):
        nsl = pl.ds(0, N)
        s = a_ref[0, nsl, :] + a_ref[1, nsl, :] + y_ref[...]
        h = jnp.maximum(s * dinv_ref[...] + b_ref[...], 0.0)
        hw = jnp.dot(h, w_ref[...], preferred_element_type=jnp.float32)
        y2_ref[...] = hw * dinv_ref[...]

    return pl.pallas_call(
        body,
        out_shape=jax.ShapeDtypeStruct((N, H), jnp.float32),
    )(acc, y, dinv, b, W2)


def _tc_final(acc, y, dinv, b, W_ih4, bih4, bhh4, W_hh4, Wfc, bfc):
    def body(a_ref, y_ref, dinv_ref, b_ref, wih_ref, bih_ref, bhh_ref,
             whh_ref, wfc_ref, bfc_ref, out_ref,
             gi_ref, gf_ref, gg_ref, go_ref, hs_ref):
        nsl = pl.ds(0, N)
        s = a_ref[0, nsl, :] + a_ref[1, nsl, :] + y_ref[...]
        h2 = jnp.maximum(s * dinv_ref[...] + b_ref[...], 0.0)
        dn = (((1,), (1,)), ((), ()))
        for k, (g_ref, scale) in enumerate((
                (gi_ref, 0.5), (gf_ref, 0.5), (gg_ref, 1.0), (go_ref, 0.5))):
            g_ref[...] = scale * (
                lax.dot_general(h2, wih_ref[k], dn,
                                preferred_element_type=jnp.float32)
                + bih_ref[k] + bhh_ref[k]
            )

        wi = (0.5 * whh_ref[0]).astype(jnp.bfloat16)
        wf = (0.5 * whh_ref[1]).astype(jnp.bfloat16)
        wg = whh_ref[2].astype(jnp.bfloat16)
        wo = (0.5 * whh_ref[3]).astype(jnp.bfloat16)

        def step(t, carry):
            h, c = carry
            hb = h.astype(jnp.bfloat16)
            ui = lax.dot_general(hb, wi, dn, preferred_element_type=jnp.float32)
            uf = lax.dot_general(hb, wf, dn, preferred_element_type=jnp.float32)
            ug = lax.dot_general(hb, wg, dn, preferred_element_type=jnp.float32)
            uo = lax.dot_general(hb, wo, dn, preferred_element_type=jnp.float32)
            row = pl.ds(t, 1)
            i = 0.5 * jnp.tanh(gi_ref[row, :] + ui) + 0.5
            f = 0.5 * jnp.tanh(gf_ref[row, :] + uf) + 0.5
            g = jnp.tanh(gg_ref[row, :] + ug)
            o = 0.5 * jnp.tanh(go_ref[row, :] + uo) + 0.5
            c = f * c + i * g
            h = o * jnp.tanh(c)
            hs_ref[row, :] = h
            return (h, c)

        h0 = jnp.zeros((1, H), jnp.float32)
        lax.fori_loop(0, N, step, (h0, h0))
        out_ref[...] = (
            lax.dot_general(hs_ref[...], wfc_ref[...], dn,
                            preferred_element_type=jnp.float32)
            + bfc_ref[...]
        )

    sd = pltpu.VMEM((N, H), jnp.float32)
    return pl.pallas_call(
        body,
        out_shape=jax.ShapeDtypeStruct((N, T), jnp.float32),
        scratch_shapes=[sd, sd, sd, sd, sd],
    )(acc, y, dinv, b, W_ih4, bih4, bhh4, W_hh4, Wfc, bfc)


def kernel(x, edge_index, W1, b1, W2, b2, W_ih, W_hh, b_ih, b_hh, Wfc, bfc):
    edges = edge_index.reshape(2, NW, NCHUNK, CH)

    deg_parts = _sc_degree(edges).reshape(NC, NPAD, 1)
    y1, dinv = _tc_first(x, W1, deg_parts)

    acc1 = _sc_message(edges, y1)
    y2 = _tc_mid(acc1, y1, dinv, b1.reshape(1, H), W2)

    acc2 = _sc_message(edges, y2)
    return _tc_final(
        acc2, y2, dinv, b2.reshape(1, H),
        W_ih.reshape(4, H, H), b_ih.reshape(4, 1, H), b_hh.reshape(4, 1, H),
        W_hh.reshape(4, H, H), Wfc, bfc.reshape(1, T),
    )


# speedup vs baseline: 1.3971x; 1.3516x over previous
import functools

import jax
import jax.numpy as jnp
from jax import lax
from jax.experimental import pallas as pl
from jax.experimental.pallas import tpu as pltpu
from jax.experimental.pallas import tpu_sc as plsc

N = 10000
D = 128
H = 32
T = 10
E = 320000

NC = 2
NS = 16
NW = NC * NS
EPW = E // NW
CH = 80
NCHUNK = EPW // CH
ROWS_PT = 640
NPAD = NS * ROWS_PT


def _mesh():
    return plsc.VectorSubcoreMesh(core_axis_name="c", subcore_axis_name="s")


@functools.partial(
    pl.kernel,
    out_type=jax.ShapeDtypeStruct((NC, NPAD), jnp.float32),
    mesh=_mesh(),
    compiler_params=pltpu.CompilerParams(use_tc_tiling_on_sc=False),
    scratch_types=[
        pltpu.VMEM((NCHUNK, CH), jnp.int32),
        pltpu.VMEM((CH,), jnp.float32),
        pltpu.VMEM((ROWS_PT,), jnp.float32),
        pltpu.VMEM_SHARED((NPAD,), jnp.float32),
        pltpu.SemaphoreType.DMA,
    ],
)
def _sc_degree(edges_hbm, out_hbm, idx_v, ones_v, buf_v, acc_sh, sem):
    cid = lax.axis_index("c")
    sid = lax.axis_index("s")
    wid = sid * NC + cid

    def fill(i, _):
        buf_v[pl.ds(i * 16, 16)] = jnp.zeros((16,), jnp.float32)
        return 0

    lax.fori_loop(0, ROWS_PT // 16, fill, 0)

    def fill1(i, _):
        ones_v[pl.ds(i * 16, 16)] = jnp.ones((16,), jnp.float32)
        return 0

    lax.fori_loop(0, CH // 16, fill1, 0)

    pltpu.sync_copy(buf_v, acc_sh.at[pl.ds(sid * ROWS_PT, ROWS_PT)])
    pltpu.sync_copy(edges_hbm.at[1, wid], idx_v)
    plsc.subcore_barrier()

    def body(j, _):
        pltpu.sync_copy(ones_v, acc_sh.at[idx_v.at[j]], add=True)
        return 0

    lax.fori_loop(0, NCHUNK, body, 0)
    plsc.subcore_barrier()

    pltpu.sync_copy(acc_sh.at[pl.ds(sid * ROWS_PT, ROWS_PT)], buf_v)
    pltpu.sync_copy(buf_v, out_hbm.at[cid, pl.ds(sid * ROWS_PT, ROWS_PT)])


@functools.partial(
    pl.kernel,
    out_type=jax.ShapeDtypeStruct((NC, NPAD, H), jnp.float32),
    mesh=_mesh(),
    compiler_params=pltpu.CompilerParams(use_tc_tiling_on_sc=False),
    scratch_types=[
        pltpu.VMEM((NCHUNK, CH), jnp.int32),
        pltpu.VMEM((NCHUNK, CH), jnp.int32),
        pltpu.VMEM((2, CH, H), jnp.float32),
        pltpu.VMEM((ROWS_PT, H), jnp.float32),
        pltpu.VMEM_SHARED((NPAD, H), jnp.float32),
        pltpu.SemaphoreType.DMA,
    ],
)
def _sc_message(edges_hbm, y_hbm, out_hbm, srcv, dstv, rows, buf, acc_sh, sem):
    cid = lax.axis_index("c")
    sid = lax.axis_index("s")
    wid = sid * NC + cid

    def fill(i, _):
        buf[i, pl.ds(0, 16)] = jnp.zeros((16,), jnp.float32)
        buf[i, pl.ds(16, 16)] = jnp.zeros((16,), jnp.float32)
        return 0

    lax.fori_loop(0, ROWS_PT, fill, 0)

    pltpu.sync_copy(buf, acc_sh.at[pl.ds(sid * ROWS_PT, ROWS_PT)])
    pltpu.sync_copy(edges_hbm.at[0, wid], srcv)
    pltpu.sync_copy(edges_hbm.at[1, wid], dstv)
    plsc.subcore_barrier()

    pltpu.async_copy(y_hbm.at[srcv.at[0]], rows.at[0], sem)

    def body(j, _):
        jm = j % 2
        pltpu.make_async_copy(y_hbm.at[srcv.at[0]], rows.at[jm], sem).wait()

        @pl.when(j + 1 < NCHUNK)
        def _():
            pltpu.async_copy(y_hbm.at[srcv.at[j + 1]], rows.at[1 - jm], sem)

        pltpu.sync_copy(rows.at[jm], acc_sh.at[dstv.at[j]], add=True)
        return 0

    lax.fori_loop(0, NCHUNK, body, 0)
    plsc.subcore_barrier()

    pltpu.sync_copy(acc_sh.at[pl.ds(sid * ROWS_PT, ROWS_PT)], buf)
    pltpu.sync_copy(buf, out_hbm.at[cid, pl.ds(sid * ROWS_PT, ROWS_PT)])


def _tc_first(x, W1, deg_parts):
    def body(x_ref, w_ref, d_ref, y_ref, dinv_ref):
        nsl = pl.ds(0, N)
        deg = d_ref[0, nsl, :] + d_ref[1, nsl, :] + 1.0
        dinv = lax.rsqrt(deg)
        xw = jnp.dot(x_ref[...], w_ref[...], preferred_element_type=jnp.float32)
        y_ref[...] = xw * dinv
        dinv_ref[...] = dinv

    return pl.pallas_call(
        body,
        out_shape=[
            jax.ShapeDtypeStruct((N, H), jnp.float32),
            jax.ShapeDtypeStruct((N, 1), jnp.float32),
        ],
    )(x, W1, deg_parts)


def _tc_mid(acc, y, dinv, b, W2):
    def body(a_ref, y_ref, dinv_ref, b_ref, w_ref, y2_ref):
        nsl = pl.ds(0, N)
        s = a_ref[0, nsl, :] + a_ref[1, nsl, :] + y_ref[...]
        h = jnp.maximum(s * dinv_ref[...] + b_ref[...], 0.0)
        hw = jnp.dot(h, w_ref[...], preferred_element_type=jnp.float32)
        y2_ref[...] = hw * dinv_ref[...]

    return pl.pallas_call(
        body,
        out_shape=jax.ShapeDtypeStruct((N, H), jnp.float32),
    )(acc, y, dinv, b, W2)


def _tc_final(acc, y, dinv, b, W_ih4, bih4, bhh4, W_hh4, Wfc, bfc):
    def body(a_ref, y_ref, dinv_ref, b_ref, wih_ref, bih_ref, bhh_ref,
             whh_ref, wfc_ref, bfc_ref, out_ref,
             gi_ref, gf_ref, gg_ref, go_ref, hs_ref):
        nsl = pl.ds(0, N)
        s = a_ref[0, nsl, :] + a_ref[1, nsl, :] + y_ref[...]
        h2 = jnp.maximum(s * dinv_ref[...] + b_ref[...], 0.0)
        dn = (((1,), (1,)), ((), ()))
        for k, (g_ref, scale) in enumerate((
                (gi_ref, 0.5), (gf_ref, 0.5), (gg_ref, 1.0), (go_ref, 0.5))):
            g_ref[...] = scale * (
                lax.dot_general(h2, wih_ref[k], dn,
                                preferred_element_type=jnp.float32)
                + bih_ref[k] + bhh_ref[k]
            )

        wi = 0.5 * whh_ref[0].T
        wf = 0.5 * whh_ref[1].T
        wg = whh_ref[2].T
        wo = 0.5 * whh_ref[3].T

        def step(t, carry):
            h, c = carry
            hcol = h.reshape(H, 1)
            ui = jnp.sum(wi * hcol, axis=0, keepdims=True)
            uf = jnp.sum(wf * hcol, axis=0, keepdims=True)
            ug = jnp.sum(wg * hcol, axis=0, keepdims=True)
            uo = jnp.sum(wo * hcol, axis=0, keepdims=True)
            row = pl.ds(t, 1)
            i = 0.5 * jnp.tanh(gi_ref[row, :] + ui) + 0.5
            f = 0.5 * jnp.tanh(gf_ref[row, :] + uf) + 0.5
            g = jnp.tanh(gg_ref[row, :] + ug)
            o = 0.5 * jnp.tanh(go_ref[row, :] + uo) + 0.5
            c = f * c + i * g
            h = o * jnp.tanh(c)
            hs_ref[row, :] = h
            return (h, c)

        h0 = jnp.zeros((1, H), jnp.float32)
        lax.fori_loop(0, N, step, (h0, h0))
        out_ref[...] = (
            lax.dot_general(hs_ref[...], wfc_ref[...], dn,
                            preferred_element_type=jnp.float32)
            + bfc_ref[...]
        )

    sd = pltpu.VMEM((N, H), jnp.float32)
    return pl.pallas_call(
        body,
        out_shape=jax.ShapeDtypeStruct((N, T), jnp.float32),
        scratch_shapes=[sd, sd, sd, sd, sd],
    )(acc, y, dinv, b, W_ih4, bih4, bhh4, W_hh4, Wfc, bfc)


def kernel(x, edge_index, W1, b1, W2, b2, W_ih, W_hh, b_ih, b_hh, Wfc, bfc):
    edges = edge_index.reshape(2, NW, NCHUNK, CH)

    deg_parts = _sc_degree(edges).reshape(NC, NPAD, 1)
    y1, dinv = _tc_first(x, W1, deg_parts)

    acc1 = _sc_message(edges, y1)
    y2 = _tc_mid(acc1, y1, dinv, b1.reshape(1, H), W2)

    acc2 = _sc_message(edges, y2)
    return _tc_final(
        acc2, y2, dinv, b2.reshape(1, H),
        W_ih.reshape(4, H, H), b_ih.reshape(4, 1, H), b_hh.reshape(4, 1, H),
        W_hh.reshape(4, H, H), Wfc, bfc.reshape(1, T),
    )


# speedup vs baseline: 1.5456x; 1.1063x over previous
import functools

import jax
import jax.numpy as jnp
from jax import lax
from jax.experimental import pallas as pl
from jax.experimental.pallas import tpu as pltpu
from jax.experimental.pallas import tpu_sc as plsc

N = 10000
D = 128
H = 32
T = 10
E = 320000

NC = 2
NS = 16
NW = NC * NS
EPW = E // NW
CH = 80
NCHUNK = EPW // CH
ROWS_PT = 640
NPAD = NS * ROWS_PT


def _mesh():
    return plsc.VectorSubcoreMesh(core_axis_name="c", subcore_axis_name="s")


@functools.partial(
    pl.kernel,
    out_type=jax.ShapeDtypeStruct((NC, NPAD), jnp.float32),
    mesh=_mesh(),
    compiler_params=pltpu.CompilerParams(use_tc_tiling_on_sc=False),
    scratch_types=[
        pltpu.VMEM((NCHUNK, CH), jnp.int32),
        pltpu.VMEM((CH,), jnp.float32),
        pltpu.VMEM((ROWS_PT,), jnp.float32),
        pltpu.VMEM_SHARED((NPAD,), jnp.float32),
        pltpu.SemaphoreType.DMA,
    ],
)
def _sc_degree(edges_hbm, out_hbm, idx_v, ones_v, buf_v, acc_sh, sem):
    cid = lax.axis_index("c")
    sid = lax.axis_index("s")
    wid = sid * NC + cid

    def fill(i, _):
        buf_v[pl.ds(i * 16, 16)] = jnp.zeros((16,), jnp.float32)
        return 0

    lax.fori_loop(0, ROWS_PT // 16, fill, 0)

    def fill1(i, _):
        ones_v[pl.ds(i * 16, 16)] = jnp.ones((16,), jnp.float32)
        return 0

    lax.fori_loop(0, CH // 16, fill1, 0)

    pltpu.sync_copy(buf_v, acc_sh.at[pl.ds(sid * ROWS_PT, ROWS_PT)])
    pltpu.sync_copy(edges_hbm.at[1, wid], idx_v)
    plsc.subcore_barrier()

    def body(j, _):
        pltpu.sync_copy(ones_v, acc_sh.at[idx_v.at[j]], add=True)
        return 0

    lax.fori_loop(0, NCHUNK, body, 0)
    plsc.subcore_barrier()

    pltpu.sync_copy(acc_sh.at[pl.ds(sid * ROWS_PT, ROWS_PT)], buf_v)
    pltpu.sync_copy(buf_v, out_hbm.at[cid, pl.ds(sid * ROWS_PT, ROWS_PT)])


SLOTS = 5
NGRP = NCHUNK // SLOTS


@functools.partial(
    pl.kernel,
    out_type=jax.ShapeDtypeStruct((NC, NPAD, H), jnp.float32),
    mesh=_mesh(),
    compiler_params=pltpu.CompilerParams(use_tc_tiling_on_sc=False),
    scratch_types=[
        pltpu.VMEM((NCHUNK, CH), jnp.int32),
        pltpu.VMEM((NCHUNK, CH), jnp.int32),
        pltpu.VMEM((SLOTS, CH, H), jnp.float32),
        pltpu.VMEM((ROWS_PT, H), jnp.float32),
        pltpu.VMEM_SHARED((NPAD, H), jnp.float32),
        pltpu.SemaphoreType.DMA,
        pltpu.SemaphoreType.DMA,
        pltpu.SemaphoreType.DMA,
        pltpu.SemaphoreType.DMA,
        pltpu.SemaphoreType.DMA,
    ],
)
def _sc_message(edges_hbm, y_hbm, out_hbm, srcv, dstv, rows, buf, acc_sh,
                sem0, sem1, sem2, sem3, sem4):
    cid = lax.axis_index("c")
    sid = lax.axis_index("s")
    wid = sid * NC + cid
    sems = (sem0, sem1, sem2, sem3, sem4)

    def fill(i, _):
        buf[i, pl.ds(0, 16)] = jnp.zeros((16,), jnp.float32)
        buf[i, pl.ds(16, 16)] = jnp.zeros((16,), jnp.float32)
        return 0

    lax.fori_loop(0, ROWS_PT, fill, 0)

    pltpu.sync_copy(buf, acc_sh.at[pl.ds(sid * ROWS_PT, ROWS_PT)])
    pltpu.sync_copy(edges_hbm.at[0, wid], srcv)
    pltpu.sync_copy(edges_hbm.at[1, wid], dstv)
    plsc.subcore_barrier()

    for s in range(SLOTS):
        pltpu.async_copy(y_hbm.at[srcv.at[s]], rows.at[s], sems[s])

    def body(gg, _):
        j0 = gg * SLOTS
        for s in range(SLOTS):
            j = j0 + s
            pltpu.make_async_copy(y_hbm.at[srcv.at[0]], rows.at[s],
                                  sems[s]).wait()
            pltpu.sync_copy(rows.at[s], acc_sh.at[dstv.at[j]], add=True)

            @pl.when(j + SLOTS < NCHUNK)
            def _():
                pltpu.async_copy(y_hbm.at[srcv.at[j + SLOTS]], rows.at[s],
                                 sems[s])
        return 0

    lax.fori_loop(0, NGRP, body, 0)
    plsc.subcore_barrier()

    pltpu.sync_copy(acc_sh.at[pl.ds(sid * ROWS_PT, ROWS_PT)], buf)
    pltpu.sync_copy(buf, out_hbm.at[cid, pl.ds(sid * ROWS_PT, ROWS_PT)])


def _tc_first(x, W1, deg_parts):
    def body(x_ref, w_ref, d_ref, y_ref, dinv_ref):
        nsl = pl.ds(0, N)
        deg = d_ref[0, nsl, :] + d_ref[1, nsl, :] + 1.0
        dinv = lax.rsqrt(deg)
        xw = jnp.dot(x_ref[...], w_ref[...], preferred_element_type=jnp.float32)
        y_ref[...] = xw * dinv
        dinv_ref[...] = dinv

    return pl.pallas_call(
        body,
        out_shape=[
            jax.ShapeDtypeStruct((N, H), jnp.float32),
            jax.ShapeDtypeStruct((N, 1), jnp.float32),
        ],
    )(x, W1, deg_parts)


def _tc_mid(acc, y, dinv, b, W2):
    def body(a_ref, y_ref, dinv_ref, b_ref, w_ref, y2_ref):
        nsl = pl.ds(0, N)
        s = a_ref[0, nsl, :] + a_ref[1, nsl, :] + y_ref[...]
        h = jnp.maximum(s * dinv_ref[...] + b_ref[...], 0.0)
        hw = jnp.dot(h, w_ref[...], preferred_element_type=jnp.float32)
        y2_ref[...] = hw * dinv_ref[...]

    return pl.pallas_call(
        body,
        out_shape=jax.ShapeDtypeStruct((N, H), jnp.float32),
    )(acc, y, dinv, b, W2)


def _tc_final(acc, y, dinv, b, W_ih4, bih4, bhh4, W_hh4, Wfc, bfc):
    def body(a_ref, y_ref, dinv_ref, b_ref, wih_ref, bih_ref, bhh_ref,
             whh_ref, wfc_ref, bfc_ref, out_ref,
             gi_ref, gf_ref, gg_ref, go_ref, hs_ref):
        nsl = pl.ds(0, N)
        s = a_ref[0, nsl, :] + a_ref[1, nsl, :] + y_ref[...]
        h2 = jnp.maximum(s * dinv_ref[...] + b_ref[...], 0.0)
        dn = (((1,), (1,)), ((), ()))
        for k, (g_ref, scale) in enumerate((
                (gi_ref, 0.5), (gf_ref, 0.5), (gg_ref, 1.0), (go_ref, 0.5))):
            g_ref[...] = scale * (
                lax.dot_general(h2, wih_ref[k], dn,
                                preferred_element_type=jnp.float32)
                + bih_ref[k] + bhh_ref[k]
            )

        wi = 0.5 * whh_ref[0].T
        wf = 0.5 * whh_ref[1].T
        wg = whh_ref[2].T
        wo = 0.5 * whh_ref[3].T

        def step(t, carry):
            h, c = carry
            hcol = h.reshape(H, 1)
            ui = jnp.sum(wi * hcol, axis=0, keepdims=True)
            uf = jnp.sum(wf * hcol, axis=0, keepdims=True)
            ug = jnp.sum(wg * hcol, axis=0, keepdims=True)
            uo = jnp.sum(wo * hcol, axis=0, keepdims=True)
            row = pl.ds(t, 1)
            i = 0.5 * jnp.tanh(gi_ref[row, :] + ui) + 0.5
            f = 0.5 * jnp.tanh(gf_ref[row, :] + uf) + 0.5
            g = jnp.tanh(gg_ref[row, :] + ug)
            o = 0.5 * jnp.tanh(go_ref[row, :] + uo) + 0.5
            c = f * c + i * g
            h = o * jnp.tanh(c)
            hs_ref[row, :] = h
            return (h, c)

        h0 = jnp.zeros((1, H), jnp.float32)
        lax.fori_loop(0, N, step, (h0, h0))
        out_ref[...] = (
            lax.dot_general(hs_ref[...], wfc_ref[...], dn,
                            preferred_element_type=jnp.float32)
            + bfc_ref[...]
        )

    sd = pltpu.VMEM((N, H), jnp.float32)
    return pl.pallas_call(
        body,
        out_shape=jax.ShapeDtypeStruct((N, T), jnp.float32),
        scratch_shapes=[sd, sd, sd, sd, sd],
    )(acc, y, dinv, b, W_ih4, bih4, bhh4, W_hh4, Wfc, bfc)


def kernel(x, edge_index, W1, b1, W2, b2, W_ih, W_hh, b_ih, b_hh, Wfc, bfc):
    edges = edge_index.reshape(2, NW, NCHUNK, CH)

    deg_parts = _sc_degree(edges).reshape(NC, NPAD, 1)
    y1, dinv = _tc_first(x, W1, deg_parts)

    acc1 = _sc_message(edges, y1)
    y2 = _tc_mid(acc1, y1, dinv, b1.reshape(1, H), W2)

    acc2 = _sc_message(edges, y2)
    return _tc_final(
        acc2, y2, dinv, b2.reshape(1, H),
        W_ih.reshape(4, H, H), b_ih.reshape(4, 1, H), b_hh.reshape(4, 1, H),
        W_hh.reshape(4, H, H), Wfc, bfc.reshape(1, T),
    )


# speedup vs baseline: 1.5467x; 1.0007x over previous
import functools

import jax
import jax.numpy as jnp
from jax import lax
from jax.experimental import pallas as pl
from jax.experimental.pallas import tpu as pltpu
from jax.experimental.pallas import tpu_sc as plsc

N = 10000
D = 128
H = 32
T = 10
E = 320000

NC = 2
NS = 16
NW = NC * NS
EPW = E // NW
CH = 80
NCHUNK = EPW // CH
ROWS_PT = 640
NPAD = NS * ROWS_PT


def _mesh():
    return plsc.VectorSubcoreMesh(core_axis_name="c", subcore_axis_name="s")


@functools.partial(
    pl.kernel,
    out_type=jax.ShapeDtypeStruct((NC, NPAD, H), jnp.float32),
    mesh=_mesh(),
    compiler_params=pltpu.CompilerParams(use_tc_tiling_on_sc=False),
    scratch_types=[
        pltpu.VMEM((NCHUNK, CH), jnp.int32),
        pltpu.VMEM((CH, H), jnp.float32),
        pltpu.VMEM((ROWS_PT, H), jnp.float32),
        pltpu.VMEM_SHARED((NPAD, H), jnp.float32),
        pltpu.SemaphoreType.DMA,
        pltpu.SemaphoreType.DMA,
        pltpu.SemaphoreType.DMA,
        pltpu.SemaphoreType.DMA,
        pltpu.SemaphoreType.DMA,
    ],
)
def _sc_degree(edges_hbm, out_hbm, idx_v, ones_v, buf_v, acc_sh,
               sem0, sem1, sem2, sem3, sem4):
    cid = lax.axis_index("c")
    sid = lax.axis_index("s")
    wid = sid * NC + cid
    sems = (sem0, sem1, sem2, sem3, sem4)

    def fill(i, _):
        buf_v[i, pl.ds(0, 16)] = jnp.zeros((16,), jnp.float32)
        buf_v[i, pl.ds(16, 16)] = jnp.zeros((16,), jnp.float32)
        return 0

    lax.fori_loop(0, ROWS_PT, fill, 0)

    def fill1(i, _):
        ones_v[i, pl.ds(0, 16)] = jnp.ones((16,), jnp.float32)
        ones_v[i, pl.ds(16, 16)] = jnp.ones((16,), jnp.float32)
        return 0

    lax.fori_loop(0, CH, fill1, 0)

    pltpu.sync_copy(buf_v, acc_sh.at[pl.ds(sid * ROWS_PT, ROWS_PT)])
    pltpu.sync_copy(edges_hbm.at[1, wid], idx_v)
    plsc.subcore_barrier()

    for s in range(SLOTS):
        pltpu.async_copy(ones_v, acc_sh.at[idx_v.at[s]], sems[s], add=True)

    def body(gg, _):
        j0 = gg * SLOTS
        for s in range(SLOTS):
            j = j0 + s
            pltpu.make_async_copy(ones_v, acc_sh.at[idx_v.at[0]],
                                  sems[s]).wait()

            @pl.when(j + SLOTS < NCHUNK)
            def _():
                pltpu.async_copy(ones_v, acc_sh.at[idx_v.at[j + SLOTS]],
                                 sems[s], add=True)
        return 0

    lax.fori_loop(0, NGRP, body, 0)
    plsc.subcore_barrier()

    pltpu.sync_copy(acc_sh.at[pl.ds(sid * ROWS_PT, ROWS_PT)], buf_v)
    pltpu.sync_copy(buf_v, out_hbm.at[cid, pl.ds(sid * ROWS_PT, ROWS_PT)])


SLOTS = 5
NGRP = NCHUNK // SLOTS


@functools.partial(
    pl.kernel,
    out_type=jax.ShapeDtypeStruct((NC, NPAD, H), jnp.float32),
    mesh=_mesh(),
    compiler_params=pltpu.CompilerParams(use_tc_tiling_on_sc=False),
    scratch_types=[
        pltpu.VMEM((NCHUNK, CH), jnp.int32),
        pltpu.VMEM((NCHUNK, CH), jnp.int32),
        pltpu.VMEM((SLOTS, CH, H), jnp.float32),
        pltpu.VMEM((ROWS_PT, H), jnp.float32),
        pltpu.VMEM_SHARED((NPAD, H), jnp.float32),
        pltpu.SemaphoreType.DMA,
        pltpu.SemaphoreType.DMA,
        pltpu.SemaphoreType.DMA,
        pltpu.SemaphoreType.DMA,
        pltpu.SemaphoreType.DMA,
    ],
)
def _sc_message(edges_hbm, y_hbm, out_hbm, srcv, dstv, rows, buf, acc_sh,
                sem0, sem1, sem2, sem3, sem4):
    cid = lax.axis_index("c")
    sid = lax.axis_index("s")
    wid = sid * NC + cid
    sems = (sem0, sem1, sem2, sem3, sem4)

    def fill(i, _):
        buf[i, pl.ds(0, 16)] = jnp.zeros((16,), jnp.float32)
        buf[i, pl.ds(16, 16)] = jnp.zeros((16,), jnp.float32)
        return 0

    lax.fori_loop(0, ROWS_PT, fill, 0)

    pltpu.sync_copy(buf, acc_sh.at[pl.ds(sid * ROWS_PT, ROWS_PT)])
    pltpu.sync_copy(edges_hbm.at[0, wid], srcv)
    pltpu.sync_copy(edges_hbm.at[1, wid], dstv)
    plsc.subcore_barrier()

    for s in range(SLOTS):
        pltpu.async_copy(y_hbm.at[srcv.at[s]], rows.at[s], sems[s])

    def body(gg, _):
        j0 = gg * SLOTS
        for s in range(SLOTS):
            j = j0 + s
            pltpu.make_async_copy(y_hbm.at[srcv.at[0]], rows.at[s],
                                  sems[s]).wait()
            pltpu.sync_copy(rows.at[s], acc_sh.at[dstv.at[j]], add=True)

            @pl.when(j + SLOTS < NCHUNK)
            def _():
                pltpu.async_copy(y_hbm.at[srcv.at[j + SLOTS]], rows.at[s],
                                 sems[s])
        return 0

    lax.fori_loop(0, NGRP, body, 0)
    plsc.subcore_barrier()

    pltpu.sync_copy(acc_sh.at[pl.ds(sid * ROWS_PT, ROWS_PT)], buf)
    pltpu.sync_copy(buf, out_hbm.at[cid, pl.ds(sid * ROWS_PT, ROWS_PT)])


def _tc_first(x, W1, deg_parts):
    def body(x_ref, w_ref, d_ref, y_ref, dinv_ref):
        nsl = pl.ds(0, N)
        deg = d_ref[0, nsl, :] + d_ref[1, nsl, :] + 1.0
        dinv = lax.rsqrt(deg)
        xw = jnp.dot(x_ref[...], w_ref[...], preferred_element_type=jnp.float32)
        y_ref[...] = xw * dinv
        dinv_ref[...] = dinv

    return pl.pallas_call(
        body,
        out_shape=[
            jax.ShapeDtypeStruct((N, H), jnp.float32),
            jax.ShapeDtypeStruct((N, H), jnp.float32),
        ],
    )(x, W1, deg_parts)


def _tc_mid(acc, y, dinv, b, W2):
    def body(a_ref, y_ref, dinv_ref, b_ref, w_ref, y2_ref):
        nsl = pl.ds(0, N)
        s = a_ref[0, nsl, :] + a_ref[1, nsl, :] + y_ref[...]
        h = jnp.maximum(s * dinv_ref[...] + b_ref[...], 0.0)
        hw = jnp.dot(h, w_ref[...], preferred_element_type=jnp.float32)
        y2_ref[...] = hw * dinv_ref[...]

    return pl.pallas_call(
        body,
        out_shape=jax.ShapeDtypeStruct((N, H), jnp.float32),
    )(acc, y, dinv, b, W2)


def _tc_final(acc, y, dinv, b, W_ih4, bih4, bhh4, W_hh4, Wfc, bfc):
    def body(a_ref, y_ref, dinv_ref, b_ref, wih_ref, bih_ref, bhh_ref,
             whh_ref, wfc_ref, bfc_ref, out_ref,
             gi_ref, gf_ref, gg_ref, go_ref, hs_ref):
        nsl = pl.ds(0, N)
        s = a_ref[0, nsl, :] + a_ref[1, nsl, :] + y_ref[...]
        h2 = jnp.maximum(s * dinv_ref[...] + b_ref[...], 0.0)
        dn = (((1,), (1,)), ((), ()))
        for k, (g_ref, scale) in enumerate((
                (gi_ref, 0.5), (gf_ref, 0.5), (gg_ref, 1.0), (go_ref, 0.5))):
            g_ref[...] = scale * (
                lax.dot_general(h2, wih_ref[k], dn,
                                preferred_element_type=jnp.float32)
                + bih_ref[k] + bhh_ref[k]
            )

        wi = 0.5 * whh_ref[0].T
        wf = 0.5 * whh_ref[1].T
        wg = whh_ref[2].T
        wo = 0.5 * whh_ref[3].T

        def step(t, carry):
            h, c = carry
            hcol = h.reshape(H, 1)
            ui = jnp.sum(wi * hcol, axis=0, keepdims=True)
            uf = jnp.sum(wf * hcol, axis=0, keepdims=True)
            ug = jnp.sum(wg * hcol, axis=0, keepdims=True)
            uo = jnp.sum(wo * hcol, axis=0, keepdims=True)
            row = pl.ds(t, 1)
            i = 0.5 * jnp.tanh(gi_ref[row, :] + ui) + 0.5
            f = 0.5 * jnp.tanh(gf_ref[row, :] + uf) + 0.5
            g = jnp.tanh(gg_ref[row, :] + ug)
            o = 0.5 * jnp.tanh(go_ref[row, :] + uo) + 0.5
            c = f * c + i * g
            h = o * jnp.tanh(c)
            hs_ref[row, :] = h
            return (h, c)

        def step2(tt, carry):
            carry = step(2 * tt, carry)
            return step(2 * tt + 1, carry)

        h0 = jnp.zeros((1, H), jnp.float32)
        lax.fori_loop(0, N // 2, step2, (h0, h0))
        out_ref[...] = (
            lax.dot_general(hs_ref[...], wfc_ref[...], dn,
                            preferred_element_type=jnp.float32)
            + bfc_ref[...]
        )

    sd = pltpu.VMEM((N, H), jnp.float32)
    return pl.pallas_call(
        body,
        out_shape=jax.ShapeDtypeStruct((N, T), jnp.float32),
        scratch_shapes=[sd, sd, sd, sd, sd],
    )(acc, y, dinv, b, W_ih4, bih4, bhh4, W_hh4, Wfc, bfc)


def kernel(x, edge_index, W1, b1, W2, b2, W_ih, W_hh, b_ih, b_hh, Wfc, bfc):
    edges = edge_index.reshape(2, NW, NCHUNK, CH)

    deg_parts = _sc_degree(edges)
    y1, dinv = _tc_first(x, W1, deg_parts)

    acc1 = _sc_message(edges, y1)
    y2 = _tc_mid(acc1, y1, dinv, b1.reshape(1, H), W2)

    acc2 = _sc_message(edges, y2)
    return _tc_final(
        acc2, y2, dinv, b2.reshape(1, H),
        W_ih.reshape(4, H, H), b_ih.reshape(4, 1, H), b_hh.reshape(4, 1, H),
        W_hh.reshape(4, H, H), Wfc, bfc.reshape(1, T),
    )
